# baseline (speedup 1.0000x reference)
import functools

import jax
import jax.numpy as jnp
from jax import lax
from jax.experimental import pallas as pl
from jax.experimental.pallas import tpu as pltpu
from jax.experimental.pallas import tpu_sc as plsc

_DOWNSAMPLE = 4
_CH = 16
_SEG_PER_W = 128
_F = 512
_NW = 32


_NV = _F // 16


def _sc_body(x_hbm, sid_hbm, wb_hbm, out_hbm,
             wb_v, sbufA, xbufA, sbufB, xbufB, acc_v,
             semAx, semAs, semBx, semBs):
    wid = lax.axis_index("s") * 2 + lax.axis_index("c")
    pltpu.sync_copy(wb_hbm.at[pl.ds(wid * 16, 16)], wb_v)
    wbv = wb_v[pl.ds(0, 16)]
    rlo = wbv[0]
    rhi = wbv[1]
    gbase = wid * _SEG_PER_W

    c0 = rlo // _CH
    nch = (rhi - 1) // _CH + 1 - c0
    odd = nch & 1
    ext_down = jnp.where((c0 > 0) & (odd == 1), 1, 0)
    c0 = c0 - ext_down
    nch = nch + odd

    def start(c, xb, sb, sx, ss):
        pltpu.async_copy(x_hbm.at[pl.ds(c * _CH, _CH)], xb, sx)
        pltpu.async_copy(sid_hbm.at[pl.ds(c * _CH, _CH)], sb, ss)

    def wait(xb, sb, sx, ss):
        pltpu.make_async_copy(x_hbm.at[pl.ds(0, _CH)], xb, sx).wait()
        pltpu.make_async_copy(sid_hbm.at[pl.ds(0, _CH)], sb, ss).wait()

    def compute(xb, sb, carry):
        gp = carry[0]
        regs = list(carry[1:])
        svec = sb[pl.ds(0, _CH)]
        for r in range(_CH):
            g = svec[r]
            l = g - gbase
            inb = (l >= 0) & (l < _SEG_PER_W)
            lc = jnp.where(inb, l, _SEG_PER_W)
            change = g != gp
            for j in range(_NV):
                xv = xb[r, pl.ds(j * 16, 16)]
                regs[j] = jnp.where(change, xv, jnp.maximum(regs[j], xv))
                acc_v[lc, pl.ds(j * 16, 16)] = regs[j]
            gp = g
        return (gp, *regs)

    carry = (jnp.int32(-1),) + tuple(
        acc_v[_SEG_PER_W, pl.ds(j * 16, 16)] for j in range(_NV))
    start(c0, xbufA, sbufA, semAx, semAs)

    def pair(i, carry):
        ca = c0 + 2 * i
        start(ca + 1, xbufB, sbufB, semBx, semBs)
        wait(xbufA, sbufA, semAx, semAs)
        carry = compute(xbufA, sbufA, carry)
        last = ca + 2 >= c0 + nch
        cna = jnp.where(last, ca, ca + 2)
        start(cna, xbufA, sbufA, semAx, semAs)
        wait(xbufB, sbufB, semBx, semBs)
        carry = compute(xbufB, sbufB, carry)
        return carry

    carry = lax.fori_loop(0, nch // 2, pair, carry)
    wait(xbufA, sbufA, semAx, semAs)
    pltpu.sync_copy(acc_v.at[pl.ds(0, _SEG_PER_W)],
                    out_hbm.at[pl.ds(gbase, _SEG_PER_W)])


def _segment_max_sc(x_flat, sid_g, wbounds, BT, P_total):
    mesh = plsc.VectorSubcoreMesh(core_axis_name="c", subcore_axis_name="s")
    f = pl.kernel(
        _sc_body,
        out_type=jax.ShapeDtypeStruct((P_total, _F), jnp.float32),
        mesh=mesh,
        scratch_types=[
            pltpu.VMEM((16,), jnp.int32),
            pltpu.VMEM((_CH,), jnp.int32),
            pltpu.VMEM((_CH, _F), jnp.float32),
            pltpu.VMEM((_CH,), jnp.int32),
            pltpu.VMEM((_CH, _F), jnp.float32),
            pltpu.VMEM((_SEG_PER_W + 1, _F), jnp.float32),
            pltpu.SemaphoreType.DMA,
            pltpu.SemaphoreType.DMA,
            pltpu.SemaphoreType.DMA,
            pltpu.SemaphoreType.DMA,
        ],
    )
    return f(x_flat, sid_g, wbounds)


def kernel(x):
    B, T, F = x.shape
    npoints = T // _DOWNSAMPLE
    aux1 = x[:, : T - 1, :]
    aux2 = x[:, 1:, :]
    aux1E = jnp.sum(aux1 * aux1, axis=2)
    aux2E = jnp.sum(aux2 * aux2, axis=2)
    dif = aux2E - aux1E
    dif_conc = jnp.concatenate([jnp.zeros((B, 1), jnp.float32), dif], axis=1)
    LT = jnp.cumsum(jnp.abs(dif_conc), axis=1)
    LT_norm = LT / LT[:, -1:]
    LT_dif = LT_norm[:, 1:] - LT_norm[:, :-1]
    _vals, indices = jax.lax.top_k(LT_dif, npoints - 1)
    rows = jnp.arange(B, dtype=jnp.int32)[:, None]
    whichs = (
        jnp.zeros((B, T - 1), jnp.float32).at[rows, indices].set(1.0))
    index_points = jnp.cumsum(
        jnp.concatenate([jnp.zeros((B, 1), jnp.float32), whichs], axis=1),
        axis=1).astype(jnp.int32)
    sid = index_points
    g = (jnp.arange(B, dtype=jnp.int32)[:, None] * npoints + sid).reshape(-1)
    starts = jnp.concatenate(
        [jnp.zeros((B, 1), jnp.int32), jnp.sort(indices, axis=1) + 1], axis=1)
    starts_g = (starts + jnp.arange(B, dtype=jnp.int32)[:, None] * T).reshape(-1)
    w_lo = starts_g.reshape(_NW, _SEG_PER_W)[:, 0]
    w_hi = jnp.concatenate(
        [w_lo[1:], jnp.array([B * T], dtype=jnp.int32)])
    wbounds = (
        jnp.zeros((_NW, 16), jnp.int32)
        .at[:, 0].set(w_lo).at[:, 1].set(w_hi).reshape(-1))
    out_flat = _segment_max_sc(
        x.reshape(B * T, F), g, wbounds, B * T, B * npoints)
    return out_flat.reshape(B, npoints, F)


# speedup vs baseline: 1.7071x; 1.7071x over previous
import functools

import jax
import jax.numpy as jnp
from jax import lax
from jax.experimental import pallas as pl
from jax.experimental.pallas import tpu as pltpu
from jax.experimental.pallas import tpu_sc as plsc

_DOWNSAMPLE = 4
_CH = 16
_SEG_PER_W = 128
_F = 512
_NW = 32


_NV = _F // 16


def _sc_body(x_hbm, sid_hbm, wb_hbm, out_hbm,
             wb_v, sbufA, xbufA, sbufB, xbufB, acc_v,
             semAx, semAs, semBx, semBs):
    wid = lax.axis_index("s") * 2 + lax.axis_index("c")
    pltpu.sync_copy(wb_hbm.at[pl.ds(wid * 16, 16)], wb_v)
    wbv = wb_v[pl.ds(0, 16)]
    rlo = wbv[0]
    rhi = wbv[1]
    gbase = wid * _SEG_PER_W

    c0 = rlo // _CH
    nch = (rhi - 1) // _CH + 1 - c0
    odd = nch & 1
    ext_down = jnp.where((c0 > 0) & (odd == 1), 1, 0)
    c0 = c0 - ext_down
    nch = nch + odd

    def start(c, xb, sb, sx, ss):
        pltpu.async_copy(x_hbm.at[pl.ds(c * _CH, _CH)], xb, sx)
        pltpu.async_copy(sid_hbm.at[pl.ds(c * _CH, _CH)], sb, ss)

    def wait(xb, sb, sx, ss):
        pltpu.make_async_copy(x_hbm.at[pl.ds(0, _CH)], xb, sx).wait()
        pltpu.make_async_copy(sid_hbm.at[pl.ds(0, _CH)], sb, ss).wait()

    def compute(xb, sb, carry):
        gp = carry[0]
        regs = list(carry[1:])
        svec = sb[pl.ds(0, _CH)]
        for r in range(_CH):
            g = svec[r]
            lp = gp - gbase
            inb = (lp >= 0) & (lp < _SEG_PER_W)
            lpc = jnp.where(inb, lp, _SEG_PER_W)
            change = g != gp

            @pl.when(change)
            def _flush(regs=regs, lpc=lpc):
                for j in range(_NV):
                    acc_v[lpc, pl.ds(j * 16, 16)] = regs[j]

            for j in range(_NV):
                xv = xb[r, pl.ds(j * 16, 16)]
                regs[j] = jnp.where(change, xv, jnp.maximum(regs[j], xv))
            gp = g
        return (gp, *regs)

    carry = (jnp.int32(-1),) + tuple(
        acc_v[_SEG_PER_W, pl.ds(j * 16, 16)] for j in range(_NV))
    start(c0, xbufA, sbufA, semAx, semAs)

    def pair(i, carry):
        ca = c0 + 2 * i
        start(ca + 1, xbufB, sbufB, semBx, semBs)
        wait(xbufA, sbufA, semAx, semAs)
        carry = compute(xbufA, sbufA, carry)
        last = ca + 2 >= c0 + nch
        cna = jnp.where(last, ca, ca + 2)
        start(cna, xbufA, sbufA, semAx, semAs)
        wait(xbufB, sbufB, semBx, semBs)
        carry = compute(xbufB, sbufB, carry)
        return carry

    carry = lax.fori_loop(0, nch // 2, pair, carry)
    wait(xbufA, sbufA, semAx, semAs)
    gp = carry[0]
    lp = gp - gbase
    inb = (lp >= 0) & (lp < _SEG_PER_W)
    lpc = jnp.where(inb, lp, _SEG_PER_W)
    for j in range(_NV):
        acc_v[lpc, pl.ds(j * 16, 16)] = carry[1 + j]
    pltpu.sync_copy(acc_v.at[pl.ds(0, _SEG_PER_W)],
                    out_hbm.at[pl.ds(gbase, _SEG_PER_W)])


def _segment_max_sc(x_flat, sid_g, wbounds, BT, P_total):
    mesh = plsc.VectorSubcoreMesh(core_axis_name="c", subcore_axis_name="s")
    f = pl.kernel(
        _sc_body,
        out_type=jax.ShapeDtypeStruct((P_total, _F), jnp.float32),
        mesh=mesh,
        scratch_types=[
            pltpu.VMEM((16,), jnp.int32),
            pltpu.VMEM((_CH,), jnp.int32),
            pltpu.VMEM((_CH, _F), jnp.float32),
            pltpu.VMEM((_CH,), jnp.int32),
            pltpu.VMEM((_CH, _F), jnp.float32),
            pltpu.VMEM((_SEG_PER_W + 1, _F), jnp.float32),
            pltpu.SemaphoreType.DMA,
            pltpu.SemaphoreType.DMA,
            pltpu.SemaphoreType.DMA,
            pltpu.SemaphoreType.DMA,
        ],
    )
    return f(x_flat, sid_g, wbounds)


def _select_body(idx_ref, g_ref, st_ref, *, T, P):
    b = pl.program_id(0)
    idx_row = idx_ref[0]
    iota_t = jax.lax.broadcasted_iota(jnp.int32, (T, P), 0)
    c2 = (idx_row < iota_t).astype(jnp.int32)
    sid_col = jnp.sum(c2, axis=1, keepdims=True)
    g_ref[0] = sid_col + b * P
    iota_p = jax.lax.broadcasted_iota(jnp.int32, (T, P), 1)
    c3 = (sid_col < iota_p).astype(jnp.int32)
    st_ref[0] = jnp.broadcast_to(
        jnp.sum(c3, axis=0, keepdims=True), (8, P))


def _selection_to_segments(indices, B, T, P):
    idx_pad = jnp.concatenate(
        [indices, jnp.full((B, 1), 2 * T, jnp.int32)], axis=1)
    idx3 = idx_pad[:, None, :]
    g_col, starts8 = pl.pallas_call(
        functools.partial(_select_body, T=T, P=P),
        grid=(B,),
        in_specs=[pl.BlockSpec((1, 1, P), lambda b: (b, 0, 0))],
        out_specs=[
            pl.BlockSpec((1, T, 1), lambda b: (b, 0, 0)),
            pl.BlockSpec((1, 8, P), lambda b: (b, 0, 0)),
        ],
        out_shape=[
            jax.ShapeDtypeStruct((B, T, 1), jnp.int32),
            jax.ShapeDtypeStruct((B, 8, P), jnp.int32),
        ],
    )(idx3)
    return g_col.reshape(B * T), starts8[:, 0, :]


def kernel(x):
    B, T, F = x.shape
    npoints = T // _DOWNSAMPLE
    aux1 = x[:, : T - 1, :]
    aux2 = x[:, 1:, :]
    aux1E = jnp.sum(aux1 * aux1, axis=2)
    aux2E = jnp.sum(aux2 * aux2, axis=2)
    dif = aux2E - aux1E
    dif_conc = jnp.concatenate([jnp.zeros((B, 1), jnp.float32), dif], axis=1)
    LT = jnp.cumsum(jnp.abs(dif_conc), axis=1)
    LT_norm = LT / LT[:, -1:]
    LT_dif = LT_norm[:, 1:] - LT_norm[:, :-1]
    _vals, indices = jax.lax.top_k(LT_dif, npoints - 1)
    g, starts = _selection_to_segments(indices, B, T, npoints)
    starts_g = (starts + jnp.arange(B, dtype=jnp.int32)[:, None] * T).reshape(-1)
    w_lo = starts_g.reshape(_NW, _SEG_PER_W)[:, 0]
    w_hi = jnp.concatenate(
        [w_lo[1:], jnp.array([B * T], dtype=jnp.int32)])
    wbounds = (
        jnp.zeros((_NW, 16), jnp.int32)
        .at[:, 0].set(w_lo).at[:, 1].set(w_hi).reshape(-1))
    out_flat = _segment_max_sc(
        x.reshape(B * T, F), g, wbounds, B * T, B * npoints)
    return out_flat.reshape(B, npoints, F)
